# baseline (speedup 1.0000x reference)
import functools

import jax
import jax.numpy as jnp
from jax import lax
from jax.experimental import pallas as pl
from jax.experimental.pallas import tpu as pltpu
from jax.experimental.pallas import tpu_sc as plsc

D_MODEL = 1024
DICT_SIZE = 4096
K = 8
VPAD = 16
LN_EPS = 1e-5

TOK_BLK = 256

NEG_CAP = float("-inf")


def _tc_body(x_ref, g_ref, b_ref, we_ref, sp_ref, vals_ref, idx_ref):
    xb = x_ref[...]
    m = jnp.mean(xb, axis=1, keepdims=True)
    xc = xb - m
    v = jnp.mean(xc * xc, axis=1, keepdims=True)
    normed = xc / jnp.sqrt(v + LN_EPS) * g_ref[...] + b_ref[...]
    logits = lax.dot_general(
        normed, we_ref[...], (((1,), (1,)), ((), ())),
        preferred_element_type=jnp.float32,
        precision=lax.Precision.DEFAULT,
    )

    HW = DICT_SIZE // 2
    Lh = logits[:, :HW]
    Rh = logits[:, HW:]
    smaxR = Rh > Lh
    wmax = jnp.where(smaxR, Rh, Lh)
    wmin = jnp.where(smaxR, Lh, Rh)
    iota_h = lax.broadcasted_iota(jnp.int32, (TOK_BLK, HW), 1)
    porig = iota_h + jnp.where(smaxR, HW, 0)
    porig0 = porig

    def _tree_max(a):
        w = a.shape[1]
        while w > 128:
            w //= 2
            a = jnp.maximum(a[:, :w], a[:, w:2 * w])
        return jnp.max(a, axis=1, keepdims=True)

    def _tree_min(a):
        w = a.shape[1]
        while w > 128:
            w //= 2
            a = jnp.minimum(a[:, :w], a[:, w:2 * w])
        return jnp.min(a, axis=1, keepdims=True)

    vals_cols = []
    idx_cols = []
    for _ in range(K):
        mj = _tree_max(wmax)
        pj = _tree_min(jnp.where(wmax == mj, porig, DICT_SIZE))
        hit = porig == pj
        wmax = jnp.where(hit, wmin, wmax)
        wmin = jnp.where(hit, NEG_CAP, wmin)
        porig = jnp.where(hit, porig ^ HW, porig)
        vals_cols.append(mj)
        idx_cols.append(pj)
    sel1 = wmin == NEG_CAP
    sel2 = wmax == NEG_CAP
    origL = porig0 < HW
    selL = sel2 | (sel1 & origL)
    selR = sel2 | (sel1 & jnp.logical_not(origL))
    sp_ref[:, :HW] = jnp.where(selL, Lh, 0.0)
    sp_ref[:, HW:] = jnp.where(selR, Rh, 0.0)
    vals_cols.append(jnp.zeros((TOK_BLK, VPAD - K), jnp.float32))
    vals_ref[...] = jnp.concatenate(vals_cols, axis=1)
    idx_ref[...] = jnp.concatenate(idx_cols, axis=1)


def _tc_encode_topk(xf, gamma, beta, W_enc):
    n_tok = xf.shape[0]
    grid = n_tok // TOK_BLK
    return pl.pallas_call(
        _tc_body,
        grid=(grid,),
        in_specs=[
            pl.BlockSpec((TOK_BLK, D_MODEL), lambda i: (i, 0)),
            pl.BlockSpec((1, D_MODEL), lambda i: (0, 0)),
            pl.BlockSpec((1, D_MODEL), lambda i: (0, 0)),
            pl.BlockSpec((DICT_SIZE, D_MODEL), lambda i: (0, 0)),
        ],
        out_specs=[
            pl.BlockSpec((TOK_BLK, DICT_SIZE), lambda i: (i, 0)),
            pl.BlockSpec((TOK_BLK, VPAD), lambda i: (i, 0)),
            pl.BlockSpec((TOK_BLK, K), lambda i: (i, 0)),
        ],
        out_shape=[
            jax.ShapeDtypeStruct((n_tok, DICT_SIZE), jnp.float32),
            jax.ShapeDtypeStruct((n_tok, VPAD), jnp.float32),
            jax.ShapeDtypeStruct((n_tok, K), jnp.int32),
        ],
    )(xf, gamma.reshape(1, D_MODEL), beta.reshape(1, D_MODEL), W_enc)


def _sc_reconstruct(vals, idx, W_dict, n_tok):
    num_cores, num_subcores = 2, 16
    nw = num_cores * num_subcores
    tpw = n_tok // nw
    mesh = plsc.VectorSubcoreMesh(
        core_axis_name="c",
        subcore_axis_name="s",
        num_cores=num_cores,
        num_subcores=num_subcores,
    )
    HD = D_MODEL // 2
    lo = lax.bitcast_convert_type(
        W_dict[:, :HD].astype(jnp.bfloat16), jnp.uint16).astype(jnp.int32)
    hi = lax.bitcast_convert_type(
        W_dict[:, HD:].astype(jnp.bfloat16), jnp.uint16).astype(jnp.int32)
    wd_bf = lo | (hi << 16)

    TB = 8
    NB = tpw // TB
    BR = TB * K

    @functools.partial(
        pl.kernel,
        out_type=jax.ShapeDtypeStruct((n_tok, D_MODEL), jnp.float32),
        mesh=mesh,
        scratch_types=[
            pltpu.VMEM((tpw * K + BR,), jnp.int32),
            pltpu.VMEM((tpw * VPAD,), jnp.float32),
            pltpu.VMEM((BR, HD), jnp.int32),
            pltpu.VMEM((BR, HD), jnp.int32),
            pltpu.VMEM((TB, D_MODEL), jnp.float32),
            pltpu.VMEM((TB, D_MODEL), jnp.float32),
            pltpu.SemaphoreType.DMA,
            pltpu.SemaphoreType.DMA,
            pltpu.SemaphoreType.DMA,
            pltpu.SemaphoreType.DMA,
        ],
    )
    def sc_kernel(vals_hbm, idx_hbm, wd_hbm, out_hbm, idx_v, vals_v,
                  rows_a, rows_b, obuf_a, obuf_b, sem_ga, sem_gb, sem_oa, sem_ob):
        c = lax.axis_index("c")
        s = lax.axis_index("s")
        wid = s * num_cores + c
        base = wid * tpw
        pltpu.sync_copy(idx_hbm.at[pl.ds(base * K, tpw * K)],
                        idx_v.at[pl.ds(0, tpw * K)])
        pltpu.sync_copy(vals_hbm.at[pl.ds(base * VPAD, tpw * VPAD)], vals_v)
        for z in range(BR // 16):
            idx_v[pl.ds(tpw * K + z * 16, 16)] = jnp.zeros((16,), jnp.int32)

        def g_idx(b):
            return idx_v.at[pl.ds(b * BR, BR)]

        def compute_batch(bt, rows_v, obuf_v):
            for tt in range(TB):
                t = bt * TB + tt
                vv = vals_v[pl.ds(t * VPAD, 16)]
                splats = [
                    vv.at[jnp.full((16,), j, jnp.int32)].get(mode="promise_in_bounds")
                    for j in range(K)
                ]

                def chunk_body(cc, _, tt=tt, splats=splats):
                    acc_e = jnp.zeros((16,), jnp.float32)
                    acc_o = jnp.zeros((16,), jnp.float32)
                    for j in range(K):
                        wi = rows_v[tt * K + j, pl.ds(cc * 16, 16)]
                        we = lax.bitcast_convert_type(wi << 16, jnp.float32)
                        wo = lax.bitcast_convert_type(
                            wi & jnp.int32(-65536), jnp.float32)
                        acc_e = acc_e + splats[j] * we
                        acc_o = acc_o + splats[j] * wo
                    obuf_v[tt, pl.ds(cc * 16, 16)] = acc_e
                    obuf_v[tt, pl.ds(HD + cc * 16, 16)] = acc_o
                    return _

                lax.fori_loop(0, HD // 16, chunk_body, None, unroll=4)

        pltpu.async_copy(wd_hbm.at[g_idx(0)], rows_a, sem_ga)

        def body(g, _):
            bA = 2 * g
            bB = 2 * g + 1
            pltpu.make_async_copy(wd_hbm.at[g_idx(bA)], rows_a, sem_ga).wait()
            pltpu.async_copy(wd_hbm.at[g_idx(bB)], rows_b, sem_gb)

            @pl.when(g > 0)
            def _wa():
                pltpu.make_async_copy(
                    obuf_a, out_hbm.at[pl.ds(base, TB)], sem_oa).wait()

            compute_batch(bA, rows_a, obuf_a)
            pltpu.async_copy(obuf_a, out_hbm.at[pl.ds(base + bA * TB, TB)], sem_oa)
            pltpu.make_async_copy(wd_hbm.at[g_idx(bB)], rows_b, sem_gb).wait()
            pltpu.async_copy(wd_hbm.at[g_idx(bA + 2)], rows_a, sem_ga)

            @pl.when(g > 0)
            def _wb():
                pltpu.make_async_copy(
                    obuf_b, out_hbm.at[pl.ds(base, TB)], sem_ob).wait()

            compute_batch(bB, rows_b, obuf_b)
            pltpu.async_copy(obuf_b, out_hbm.at[pl.ds(base + bB * TB, TB)], sem_ob)
            return _

        lax.fori_loop(0, NB // 2, body, None)
        pltpu.make_async_copy(wd_hbm.at[g_idx(NB)], rows_a, sem_ga).wait()
        pltpu.make_async_copy(obuf_a, out_hbm.at[pl.ds(base, TB)], sem_oa).wait()
        pltpu.make_async_copy(obuf_b, out_hbm.at[pl.ds(base, TB)], sem_ob).wait()

    return sc_kernel(vals.reshape(-1), idx.reshape(-1), wd_bf)


def kernel(x, gamma, beta, W_enc, W_dict):
    b, t, d = x.shape
    n_tok = b * t
    xf = x.reshape(n_tok, d)
    sparse, vals, idx = _tc_encode_topk(xf, gamma, beta, W_enc)
    recon = _sc_reconstruct(vals, idx, W_dict, n_tok)
    return (recon.reshape(b, t, d), sparse.reshape(b, t, DICT_SIZE))


# speedup vs baseline: 1.1224x; 1.1224x over previous
import functools

import jax
import jax.numpy as jnp
from jax import lax
from jax.experimental import pallas as pl
from jax.experimental.pallas import tpu as pltpu
from jax.experimental.pallas import tpu_sc as plsc

D_MODEL = 1024
DICT_SIZE = 4096
K = 8
VPAD = 16
LN_EPS = 1e-5

TOK_BLK = 256

NEG_CAP = float("-inf")


def _tc_body(x_ref, g_ref, b_ref, we_ref, sp_ref, vals_ref, idx_ref):
    xb = x_ref[...]
    m = jnp.mean(xb, axis=1, keepdims=True)
    xc = xb - m
    v = jnp.mean(xc * xc, axis=1, keepdims=True)
    normed = xc / jnp.sqrt(v + LN_EPS) * g_ref[...] + b_ref[...]
    logits = lax.dot_general(
        normed, we_ref[...], (((1,), (1,)), ((), ())),
        preferred_element_type=jnp.float32,
        precision=lax.Precision.DEFAULT,
    )

    HW = DICT_SIZE // 2
    Lh = logits[:, :HW]
    Rh = logits[:, HW:]
    smaxR = Rh > Lh
    wmax = jnp.where(smaxR, Rh, Lh)
    wmin = jnp.where(smaxR, Lh, Rh)
    iota_h = lax.broadcasted_iota(jnp.int32, (TOK_BLK, HW), 1)
    porig = iota_h + jnp.where(smaxR, HW, 0)
    porig0 = porig

    def _tree_max(a):
        w = a.shape[1]
        while w > 128:
            w //= 2
            a = jnp.maximum(a[:, :w], a[:, w:2 * w])
        return jnp.max(a, axis=1, keepdims=True)

    def _tree_min(a):
        w = a.shape[1]
        while w > 128:
            w //= 2
            a = jnp.minimum(a[:, :w], a[:, w:2 * w])
        return jnp.min(a, axis=1, keepdims=True)

    vals_cols = []
    idx_cols = []
    for _ in range(K):
        mj = _tree_max(wmax)
        pj = _tree_min(jnp.where(wmax == mj, porig, DICT_SIZE))
        hit = porig == pj
        wmax = jnp.where(hit, wmin, wmax)
        wmin = jnp.where(hit, NEG_CAP, wmin)
        porig = jnp.where(hit, porig ^ HW, porig)
        vals_cols.append(mj)
        idx_cols.append(pj)
    sel1 = wmin == NEG_CAP
    sel2 = wmax == NEG_CAP
    origL = porig0 < HW
    selL = sel2 | (sel1 & origL)
    selR = sel2 | (sel1 & jnp.logical_not(origL))
    sp_ref[:, :HW] = jnp.where(selL, Lh, 0.0)
    sp_ref[:, HW:] = jnp.where(selR, Rh, 0.0)
    vals_cols.append(jnp.zeros((TOK_BLK, VPAD - K), jnp.float32))
    vals_ref[...] = jnp.concatenate(vals_cols, axis=1)
    idx_ref[...] = jnp.concatenate(idx_cols, axis=1)


def _tc_encode_topk(xf, gamma, beta, W_enc):
    n_tok = xf.shape[0]
    grid = n_tok // TOK_BLK
    return pl.pallas_call(
        _tc_body,
        grid=(grid,),
        in_specs=[
            pl.BlockSpec((TOK_BLK, D_MODEL), lambda i: (i, 0)),
            pl.BlockSpec((1, D_MODEL), lambda i: (0, 0)),
            pl.BlockSpec((1, D_MODEL), lambda i: (0, 0)),
            pl.BlockSpec((DICT_SIZE, D_MODEL), lambda i: (0, 0)),
        ],
        out_specs=[
            pl.BlockSpec((TOK_BLK, DICT_SIZE), lambda i: (i, 0)),
            pl.BlockSpec((TOK_BLK, VPAD), lambda i: (i, 0)),
            pl.BlockSpec((TOK_BLK, K), lambda i: (i, 0)),
        ],
        out_shape=[
            jax.ShapeDtypeStruct((n_tok, DICT_SIZE), jnp.float32),
            jax.ShapeDtypeStruct((n_tok, VPAD), jnp.float32),
            jax.ShapeDtypeStruct((n_tok, K), jnp.int32),
        ],
    )(xf, gamma.reshape(1, D_MODEL), beta.reshape(1, D_MODEL), W_enc)


def _sc_reconstruct(vals, idx, W_dict, n_tok):
    num_cores, num_subcores = 2, 16
    nw = num_cores * num_subcores
    tpw = n_tok // nw
    mesh = plsc.VectorSubcoreMesh(
        core_axis_name="c",
        subcore_axis_name="s",
        num_cores=num_cores,
        num_subcores=num_subcores,
    )
    HD = D_MODEL // 2
    lo = lax.bitcast_convert_type(
        W_dict[:, :HD].astype(jnp.bfloat16), jnp.uint16).astype(jnp.int32)
    hi = lax.bitcast_convert_type(
        W_dict[:, HD:].astype(jnp.bfloat16), jnp.uint16).astype(jnp.int32)
    wd_bf = lo | (hi << 16)

    TB = 8
    NB = tpw // TB
    BR = TB * K

    @functools.partial(
        pl.kernel,
        out_type=jax.ShapeDtypeStruct((n_tok, D_MODEL), jnp.float32),
        mesh=mesh,
        scratch_types=[
            pltpu.VMEM((tpw * K + BR,), jnp.int32),
            pltpu.VMEM((tpw * VPAD,), jnp.float32),
            pltpu.VMEM((BR, HD), jnp.int32),
            pltpu.VMEM((BR, HD), jnp.int32),
            pltpu.VMEM((TB, D_MODEL), jnp.float32),
            pltpu.VMEM((TB, D_MODEL), jnp.float32),
            pltpu.SemaphoreType.DMA,
            pltpu.SemaphoreType.DMA,
            pltpu.SemaphoreType.DMA,
            pltpu.SemaphoreType.DMA,
        ],
    )
    def sc_kernel(vals_hbm, idx_hbm, wd_hbm, out_hbm, idx_v, vals_v,
                  rows_a, rows_b, obuf_a, obuf_b, sem_ga, sem_gb, sem_oa, sem_ob):
        c = lax.axis_index("c")
        s = lax.axis_index("s")
        wid = s * num_cores + c
        base = wid * tpw
        pltpu.sync_copy(idx_hbm.at[pl.ds(base * K, tpw * K)],
                        idx_v.at[pl.ds(0, tpw * K)])
        pltpu.sync_copy(vals_hbm.at[pl.ds(base * VPAD, tpw * VPAD)], vals_v)
        for z in range(BR // 16):
            idx_v[pl.ds(tpw * K + z * 16, 16)] = jnp.zeros((16,), jnp.int32)

        def g_idx(b):
            return idx_v.at[pl.ds(b * BR, BR)]

        def compute_batch(bt, rows_v, obuf_v):
            for tt in range(TB):
                t = bt * TB + tt
                vv = vals_v[pl.ds(t * VPAD, 16)]
                splats = [
                    vv.at[jnp.full((16,), j, jnp.int32)].get(mode="promise_in_bounds")
                    for j in range(K)
                ]

                @functools.partial(plsc.parallel_loop, 0, HD // 16, unroll=4)
                def chunk_body(cc, tt=tt, splats=splats):
                    acc_e = jnp.zeros((16,), jnp.float32)
                    acc_o = jnp.zeros((16,), jnp.float32)
                    for j in range(K):
                        wi = rows_v[tt * K + j, pl.ds(cc * 16, 16)]
                        we = lax.bitcast_convert_type(wi << 16, jnp.float32)
                        wo = lax.bitcast_convert_type(wi, jnp.float32)
                        acc_e = acc_e + splats[j] * we
                        acc_o = acc_o + splats[j] * wo
                    obuf_v[tt, pl.ds(cc * 16, 16)] = acc_e
                    obuf_v[tt, pl.ds(HD + cc * 16, 16)] = acc_o

        pltpu.async_copy(wd_hbm.at[g_idx(0)], rows_a, sem_ga)

        def body(g, _):
            bA = 2 * g
            bB = 2 * g + 1
            pltpu.make_async_copy(wd_hbm.at[g_idx(bA)], rows_a, sem_ga).wait()
            pltpu.async_copy(wd_hbm.at[g_idx(bB)], rows_b, sem_gb)

            @pl.when(g > 0)
            def _wa():
                pltpu.make_async_copy(
                    obuf_a, out_hbm.at[pl.ds(base, TB)], sem_oa).wait()

            compute_batch(bA, rows_a, obuf_a)
            pltpu.async_copy(obuf_a, out_hbm.at[pl.ds(base + bA * TB, TB)], sem_oa)
            pltpu.make_async_copy(wd_hbm.at[g_idx(bB)], rows_b, sem_gb).wait()
            pltpu.async_copy(wd_hbm.at[g_idx(bA + 2)], rows_a, sem_ga)

            @pl.when(g > 0)
            def _wb():
                pltpu.make_async_copy(
                    obuf_b, out_hbm.at[pl.ds(base, TB)], sem_ob).wait()

            compute_batch(bB, rows_b, obuf_b)
            pltpu.async_copy(obuf_b, out_hbm.at[pl.ds(base + bB * TB, TB)], sem_ob)
            return _

        lax.fori_loop(0, NB // 2, body, None)
        pltpu.make_async_copy(wd_hbm.at[g_idx(NB)], rows_a, sem_ga).wait()
        pltpu.make_async_copy(obuf_a, out_hbm.at[pl.ds(base, TB)], sem_oa).wait()
        pltpu.make_async_copy(obuf_b, out_hbm.at[pl.ds(base, TB)], sem_ob).wait()

    return sc_kernel(vals.reshape(-1), idx.reshape(-1), wd_bf)


def kernel(x, gamma, beta, W_enc, W_dict):
    b, t, d = x.shape
    n_tok = b * t
    xf = x.reshape(n_tok, d)
    sparse, vals, idx = _tc_encode_topk(xf, gamma, beta, W_enc)
    recon = _sc_reconstruct(vals, idx, W_dict, n_tok)
    return (recon.reshape(b, t, d), sparse.reshape(b, t, DICT_SIZE))
